# baseline (speedup 1.0000x reference)
import functools

import jax
import jax.numpy as jnp
from jax import lax
from jax.experimental import pallas as pl
from jax.experimental.pallas import tpu as pltpu
from jax.experimental.pallas import tpu_sc as plsc

EMB_DIM = 32
NUM_EMB = 8192
ROWS = 4096
BLK = 1024
NBLK = ROWS // BLK
CHK = 128
TOTAL = 16 * 8192
LOSS_SCALE = 1.25 / TOTAL


def _argmin_body(x_ref, w_ref, idx_ref, loss_ref, wsq_ref, wn2_ref, fiota_ref,
                 mm_ref):
    i = pl.program_id(0)

    @pl.when(i == 0)
    def _init():
        w = w_ref[...]
        wsq_ref[...] = jnp.sum(w * w, axis=1).reshape(1, NUM_EMB)
        wn2_ref[...] = w * (-2.0)
        fiota_ref[...] = lax.broadcasted_iota(
            jnp.int32, (1, NUM_EMB), 1).astype(jnp.float32)

    xb = x_ref[...]
    xsq = jnp.sum(xb * xb, axis=1, keepdims=True)
    mmn2 = lax.dot_general(xb, wn2_ref[...], (((1,), (1,)), ((), ())),
                           preferred_element_type=jnp.float32)
    mm_ref[...] = mmn2

    def chunk_step(c, carry):
        rmin, rcid = carry
        off = c * CHK
        d_c = (xsq + wsq_ref[:, pl.ds(off, CHK)]) + mm_ref[:, pl.ds(off, CHK)]
        upd = d_c < rmin
        cf = c.astype(jnp.float32)
        return jnp.minimum(rmin, d_c), jnp.where(upd, cf, rcid)

    rmin0 = jnp.full((BLK, CHK), jnp.inf, jnp.float32)
    rcid0 = jnp.zeros((BLK, CHK), jnp.float32)
    rmin, rcid = lax.fori_loop(0, NUM_EMB // CHK, chunk_step, (rmin0, rcid0))

    gmin = jnp.min(rmin, axis=1, keepdims=True)
    jf = rcid * jnp.float32(CHK) + fiota_ref[:, :CHK]
    idxf = jnp.min(jnp.where(rmin == gmin, jf, jnp.float32(1e9)), axis=1)
    idx_ref[...] = idxf.astype(jnp.int32)
    partial = jnp.sum(gmin)
    total = jnp.where(i == 0, partial, loss_ref[0, 0] + partial)
    loss_ref[0, 0] = jnp.where(i == NBLK - 1, total * LOSS_SCALE, total)


def _argmin_call(flat_x, W):
    return pl.pallas_call(
        _argmin_body,
        grid=(NBLK,),
        in_specs=[
            pl.BlockSpec((BLK, EMB_DIM), lambda i: (i, 0)),
            pl.BlockSpec((NUM_EMB, EMB_DIM), lambda i: (0, 0)),
        ],
        out_specs=[
            pl.BlockSpec((BLK,), lambda i: (i,)),
            pl.BlockSpec(memory_space=pltpu.SMEM),
        ],
        out_shape=[
            jax.ShapeDtypeStruct((ROWS,), jnp.int32),
            jax.ShapeDtypeStruct((1, 1), jnp.float32),
        ],
        scratch_shapes=[
            pltpu.VMEM((1, NUM_EMB), jnp.float32),
            pltpu.VMEM((NUM_EMB, EMB_DIM), jnp.float32),
            pltpu.VMEM((1, NUM_EMB), jnp.float32),
            pltpu.VMEM((BLK, NUM_EMB), jnp.float32),
        ],
    )(flat_x, W)


_NC = 1
_NS = 16
_NW = _NC * _NS
_BPW = ROWS // _NW


@functools.cache
def _get_sc_gather():
    @functools.partial(
        pl.kernel,
        out_type=jax.ShapeDtypeStruct((ROWS, EMB_DIM), jnp.float32),
        mesh=plsc.VectorSubcoreMesh(core_axis_name="c", subcore_axis_name="s",
                                    num_cores=_NC, num_subcores=_NS),
        scratch_types=[
            pltpu.VMEM((_BPW,), jnp.int32),
            pltpu.VMEM((_BPW, EMB_DIM), jnp.float32),
            pltpu.SemaphoreType.DMA,
        ],
        compiler_params=pltpu.CompilerParams(use_tc_tiling_on_sc=False),
    )
    def _sc_gather(table_hbm, idx_hbm, out_hbm, idx_v, rows_v, sem):
        wid = lax.axis_index("s") * _NC + lax.axis_index("c")
        base = wid * _BPW
        pltpu.sync_copy(idx_hbm.at[pl.ds(base, _BPW)], idx_v)
        pltpu.async_copy(table_hbm.at[idx_v], rows_v, sem).wait()
        pltpu.sync_copy(rows_v, out_hbm.at[pl.ds(base, _BPW)])

    return _sc_gather


def kernel(x, W):
    flat_x = x.reshape(ROWS, EMB_DIM)
    idx, loss2 = _argmin_call(flat_x, W)
    q_flat = _get_sc_gather()(W, idx)
    quantized = q_flat.reshape(x.shape)
    return (quantized, loss2[0, 0])


# speedup vs baseline: 1.0103x; 1.0103x over previous
import functools

import jax
import jax.numpy as jnp
from jax import lax
from jax.experimental import pallas as pl
from jax.experimental.pallas import tpu as pltpu
from jax.experimental.pallas import tpu_sc as plsc

EMB_DIM = 32
NUM_EMB = 8192
ROWS = 4096
BLK = 1024
NBLK = ROWS // BLK
CHK = 512
TOTAL = 16 * 8192
LOSS_SCALE = 1.25 / TOTAL


def _argmin_body(x_ref, w_ref, idx_ref, loss_ref, wsq_ref, wn2_ref, fiota_ref,
                 mm_ref):
    i = pl.program_id(0)

    @pl.when(i == 0)
    def _init():
        w = w_ref[...]
        wsq_ref[...] = jnp.sum(w * w, axis=1).reshape(1, NUM_EMB)
        wn2_ref[...] = w * (-2.0)
        fiota_ref[...] = lax.broadcasted_iota(
            jnp.int32, (1, NUM_EMB), 1).astype(jnp.float32)

    xb = x_ref[...]
    xsq = jnp.sum(xb * xb, axis=1, keepdims=True)
    mmn2 = lax.dot_general(xb, wn2_ref[...], (((1,), (1,)), ((), ())),
                           preferred_element_type=jnp.float32)
    mm_ref[...] = mmn2

    def chunk_step(c, carry):
        rmin, rcid = carry
        off = c * CHK
        d_c = (xsq + wsq_ref[:, pl.ds(off, CHK)]) + mm_ref[:, pl.ds(off, CHK)]
        upd = d_c < rmin
        cf = c.astype(jnp.float32)
        return jnp.minimum(rmin, d_c), jnp.where(upd, cf, rcid)

    rmin0 = jnp.full((BLK, CHK), jnp.inf, jnp.float32)
    rcid0 = jnp.zeros((BLK, CHK), jnp.float32)
    rmin, rcid = lax.fori_loop(0, NUM_EMB // CHK, chunk_step, (rmin0, rcid0))

    gmin = jnp.min(rmin, axis=1, keepdims=True)
    jf = rcid * jnp.float32(CHK) + fiota_ref[:, :CHK]
    idxf = jnp.min(jnp.where(rmin == gmin, jf, jnp.float32(1e9)), axis=1)
    idx_ref[...] = idxf.astype(jnp.int32)
    partial = jnp.sum(gmin)
    total = jnp.where(i == 0, partial, loss_ref[0, 0] + partial)
    loss_ref[0, 0] = jnp.where(i == NBLK - 1, total * LOSS_SCALE, total)


def _argmin_call(flat_x, W):
    return pl.pallas_call(
        _argmin_body,
        grid=(NBLK,),
        in_specs=[
            pl.BlockSpec((BLK, EMB_DIM), lambda i: (i, 0)),
            pl.BlockSpec((NUM_EMB, EMB_DIM), lambda i: (0, 0)),
        ],
        out_specs=[
            pl.BlockSpec((BLK,), lambda i: (i,)),
            pl.BlockSpec(memory_space=pltpu.SMEM),
        ],
        out_shape=[
            jax.ShapeDtypeStruct((ROWS,), jnp.int32),
            jax.ShapeDtypeStruct((1, 1), jnp.float32),
        ],
        scratch_shapes=[
            pltpu.VMEM((1, NUM_EMB), jnp.float32),
            pltpu.VMEM((NUM_EMB, EMB_DIM), jnp.float32),
            pltpu.VMEM((1, NUM_EMB), jnp.float32),
            pltpu.VMEM((BLK, NUM_EMB), jnp.float32),
        ],
    )(flat_x, W)


_NC = 1
_NS = 16
_NW = _NC * _NS
_BPW = ROWS // _NW


@functools.cache
def _get_sc_gather():
    @functools.partial(
        pl.kernel,
        out_type=jax.ShapeDtypeStruct((ROWS, EMB_DIM), jnp.float32),
        mesh=plsc.VectorSubcoreMesh(core_axis_name="c", subcore_axis_name="s",
                                    num_cores=_NC, num_subcores=_NS),
        scratch_types=[
            pltpu.VMEM((_BPW,), jnp.int32),
            pltpu.VMEM((_BPW, EMB_DIM), jnp.float32),
            pltpu.SemaphoreType.DMA,
        ],
        compiler_params=pltpu.CompilerParams(use_tc_tiling_on_sc=False),
    )
    def _sc_gather(table_hbm, idx_hbm, out_hbm, idx_v, rows_v, sem):
        wid = lax.axis_index("s") * _NC + lax.axis_index("c")
        base = wid * _BPW
        pltpu.sync_copy(idx_hbm.at[pl.ds(base, _BPW)], idx_v)
        pltpu.async_copy(table_hbm.at[idx_v], rows_v, sem).wait()
        pltpu.sync_copy(rows_v, out_hbm.at[pl.ds(base, _BPW)])

    return _sc_gather


def kernel(x, W):
    flat_x = x.reshape(ROWS, EMB_DIM)
    idx, loss2 = _argmin_call(flat_x, W)
    q_flat = _get_sc_gather()(W, idx)
    quantized = q_flat.reshape(x.shape)
    return (quantized, loss2[0, 0])


# speedup vs baseline: 2.0247x; 2.0040x over previous
import functools

import jax
import jax.numpy as jnp
from jax import lax
from jax.experimental import pallas as pl
from jax.experimental.pallas import tpu as pltpu
from jax.experimental.pallas import tpu_sc as plsc

EMB_DIM = 32
NUM_EMB = 8192
ROWS = 4096
BLK = 1024
NBLK = ROWS // BLK
CHK = 512
TOTAL = 16 * 8192
LOSS_SCALE = 1.25 / TOTAL


def _argmin_body(x_ref, w_ref, idx_ref, loss_ref, wsq_ref, wn2_ref, fiota_ref,
                 mm_ref):
    i = pl.program_id(0)

    @pl.when(i == 0)
    def _init():
        w = w_ref[...]
        wsq_ref[...] = jnp.sum(w * w, axis=1).reshape(1, NUM_EMB)
        wn2_ref[...] = w * (-2.0)
        fiota_ref[...] = lax.broadcasted_iota(
            jnp.int32, (1, NUM_EMB), 1).astype(jnp.float32)

    xb = x_ref[...]
    xsq = jnp.sum(xb * xb, axis=1, keepdims=True)
    mmn2 = lax.dot_general(xb, wn2_ref[...], (((1,), (1,)), ((), ())),
                           preferred_element_type=jnp.float32)
    mm_ref[...] = mmn2

    rmin = (xsq + wsq_ref[:, :CHK]) + mm_ref[:, :CHK]
    rcid = jnp.zeros((BLK, CHK), jnp.float32)
    for c in range(1, NUM_EMB // CHK):
        off = c * CHK
        d_c = (xsq + wsq_ref[:, off:off + CHK]) + mm_ref[:, off:off + CHK]
        upd = d_c < rmin
        rmin = jnp.minimum(rmin, d_c)
        rcid = jnp.where(upd, jnp.float32(c), rcid)

    gmin = jnp.min(rmin, axis=1, keepdims=True)
    jf = rcid * jnp.float32(CHK) + fiota_ref[:, :CHK]
    idxf = jnp.min(jnp.where(rmin == gmin, jf, jnp.float32(1e9)), axis=1)
    idx_ref[...] = idxf.astype(jnp.int32)
    partial = jnp.sum(gmin)
    total = jnp.where(i == 0, partial, loss_ref[0, 0] + partial)
    loss_ref[0, 0] = jnp.where(i == NBLK - 1, total * LOSS_SCALE, total)


def _argmin_call(flat_x, W):
    return pl.pallas_call(
        _argmin_body,
        grid=(NBLK,),
        in_specs=[
            pl.BlockSpec((BLK, EMB_DIM), lambda i: (i, 0)),
            pl.BlockSpec((NUM_EMB, EMB_DIM), lambda i: (0, 0)),
        ],
        out_specs=[
            pl.BlockSpec((BLK,), lambda i: (i,)),
            pl.BlockSpec(memory_space=pltpu.SMEM),
        ],
        out_shape=[
            jax.ShapeDtypeStruct((ROWS,), jnp.int32),
            jax.ShapeDtypeStruct((1, 1), jnp.float32),
        ],
        scratch_shapes=[
            pltpu.VMEM((1, NUM_EMB), jnp.float32),
            pltpu.VMEM((NUM_EMB, EMB_DIM), jnp.float32),
            pltpu.VMEM((1, NUM_EMB), jnp.float32),
            pltpu.VMEM((BLK, NUM_EMB), jnp.float32),
        ],
    )(flat_x, W)


_NC = 1
_NS = 16
_NW = _NC * _NS
_BPW = ROWS // _NW


@functools.cache
def _get_sc_gather():
    @functools.partial(
        pl.kernel,
        out_type=jax.ShapeDtypeStruct((ROWS, EMB_DIM), jnp.float32),
        mesh=plsc.VectorSubcoreMesh(core_axis_name="c", subcore_axis_name="s",
                                    num_cores=_NC, num_subcores=_NS),
        scratch_types=[
            pltpu.VMEM((_BPW,), jnp.int32),
            pltpu.VMEM((_BPW, EMB_DIM), jnp.float32),
            pltpu.SemaphoreType.DMA,
        ],
        compiler_params=pltpu.CompilerParams(use_tc_tiling_on_sc=False),
    )
    def _sc_gather(table_hbm, idx_hbm, out_hbm, idx_v, rows_v, sem):
        wid = lax.axis_index("s") * _NC + lax.axis_index("c")
        base = wid * _BPW
        pltpu.sync_copy(idx_hbm.at[pl.ds(base, _BPW)], idx_v)
        pltpu.async_copy(table_hbm.at[idx_v], rows_v, sem).wait()
        pltpu.sync_copy(rows_v, out_hbm.at[pl.ds(base, _BPW)])

    return _sc_gather


def kernel(x, W):
    flat_x = x.reshape(ROWS, EMB_DIM)
    idx, loss2 = _argmin_call(flat_x, W)
    q_flat = _get_sc_gather()(W, idx)
    quantized = q_flat.reshape(x.shape)
    return (quantized, loss2[0, 0])
